# baseline (speedup 1.0000x reference)
import functools
import random

import numpy as np
import jax
import jax.numpy as jnp
from jax import lax
from jax.experimental import pallas as pl
from jax.experimental.pallas import tpu as pltpu
from jax.experimental.pallas import tpu_sc as plsc


def _perm_indices(size: int) -> np.ndarray:
    h = int(np.sqrt(size))
    pad = h // 7

    def valid(idx):
        i, j = idx // h, idx % h
        return not (j < pad or i >= h - pad or j >= h - pad)

    cands = [idx for idx in range(size) if valid(idx)]
    rng = random.Random(0)
    return np.array(rng.sample(cands, len(cands)), dtype=np.int32)


_NW = 32
_L = 128


def kernel(x):
    B, S, D = x.shape
    perm = _perm_indices(S)
    K = perm.shape[0]
    ST = S // _L

    sts = sorted(set(int(p) // _L for p in perm))
    groups = {st: np.flatnonzero(perm // _L == st) for st in sts}
    gsz = len(groups[sts[0]])
    assert all(len(g) == gsz for g in groups.values()) and gsz % 16 == 0
    nvec = gsz // 16

    ntasks = len(sts) * D
    assert ntasks % _NW == 0
    tpw = ntasks // _NW

    in_idx = np.zeros((ntasks, _L), dtype=np.int32)
    out_idx = np.zeros((ntasks, _L), dtype=np.int32)
    lanes = np.zeros((ntasks, _L), dtype=np.int32)
    b_arange = np.arange(B, dtype=np.int32)
    for g in range(ntasks):
        st, d = sts[g // D], g % D
        ks = groups[st]
        in_idx[g] = (b_arange * D + d) * ST + st
        out_idx[g, :gsz] = ks * D + d
        lanes[g, :gsz] = perm[ks] % _L
    in_idx = jnp.asarray(in_idx.reshape(_NW, tpw, _L))
    out_idx = jnp.asarray(out_idx.reshape(_NW, tpw, _L))
    lanes = jnp.asarray(lanes.reshape(_NW, tpw, _L))

    mesh = plsc.VectorSubcoreMesh(core_axis_name="c", subcore_axis_name="s")

    @functools.partial(
        pl.kernel,
        mesh=mesh,
        out_type=jax.ShapeDtypeStruct((K * D, B), jnp.float32),
        scratch_types=[
            pltpu.VMEM((tpw, _L), jnp.int32),
            pltpu.VMEM((tpw, _L), jnp.int32),
            pltpu.VMEM((tpw, _L), jnp.int32),
            pltpu.VMEM((2, _L, _L), jnp.float32),
            pltpu.VMEM((2, gsz, B), jnp.float32),
            pltpu.SemaphoreType.DMA,
            pltpu.SemaphoreType.DMA,
        ],
        compiler_params=pltpu.CompilerParams(needs_layout_passes=False),
    )
    def gather_t(x_hbm, iidx_hbm, oidx_hbm, lane_hbm, out_hbm,
                 iidx_v, oidx_v, lane_v, stage, obuf, gsem, ssem):
        w = lax.axis_index("s") * 2 + lax.axis_index("c")
        pltpu.sync_copy(iidx_hbm.at[w], iidx_v)
        pltpu.sync_copy(oidx_hbm.at[w], oidx_v)
        pltpu.sync_copy(lane_hbm.at[w], lane_v)

        row_sets = [lax.iota(jnp.int32, 16) + 16 * i for i in range(nvec)]

        def transpose_cols(j, bf):
            col_sets = [lane_v[j, pl.ds(16 * i, 16)] for i in range(nvec)]

            def body(b, carry):
                bsplat = jnp.full((16,), b, jnp.int32)
                for i in range(nvec):
                    v = plsc.load_gather(stage.at[bf], [bsplat, col_sets[i]])
                    plsc.store_scatter(obuf.at[bf], [row_sets[i], bsplat], v)
                return carry

            lax.fori_loop(0, _L, body, 0)

        def start_gather(j, bf):
            return pltpu.async_copy(x_hbm.at[iidx_v.at[j]], stage.at[bf], gsem)

        def start_scatter(j, bf):
            cps = []
            for i in range(nvec):
                rows = oidx_v[j, pl.ds(16 * i, 16)]
                cps.append(pltpu.async_copy(
                    obuf.at[bf, pl.ds(16 * i, 16)], out_hbm.at[rows], ssem))
            return cps

        g_cp = [None, None]
        s_cp = [None, None]
        g_cp[0] = start_gather(0, 0)
        for j in range(tpw):
            bf = j % 2
            if j + 1 < tpw:
                g_cp[1 - bf] = start_gather(j + 1, 1 - bf)
            g_cp[bf].wait()
            if s_cp[bf] is not None:
                for cp in s_cp[bf]:
                    cp.wait()
            transpose_cols(j, bf)
            s_cp[bf] = start_scatter(j, bf)
        for cp in s_cp[(tpw - 1) % 2]:
            cp.wait()

    xv = jnp.transpose(x, (0, 2, 1)).reshape(B * D * ST, _L)
    o_view = gather_t(xv, in_idx, out_idx, lanes)
    return o_view.reshape(K, D, B).transpose(2, 0, 1)


# speedup vs baseline: 1.5906x; 1.5906x over previous
import functools
import random

import numpy as np
import jax
import jax.numpy as jnp
from jax import lax
from jax.experimental import pallas as pl
from jax.experimental.pallas import tpu as pltpu
from jax.experimental.pallas import tpu_sc as plsc


def _perm_indices(size: int) -> np.ndarray:
    h = int(np.sqrt(size))
    pad = h // 7

    def valid(idx):
        i, j = idx // h, idx % h
        return not (j < pad or i >= h - pad or j >= h - pad)

    cands = [idx for idx in range(size) if valid(idx)]
    rng = random.Random(0)
    return np.array(rng.sample(cands, len(cands)), dtype=np.int32)


_NW = 32
_L = 128


def kernel(x):
    B, S, D = x.shape
    perm = _perm_indices(S)
    K = perm.shape[0]
    ST = S // _L

    sts = sorted(set(int(p) // _L for p in perm))
    nst = len(sts)
    groups = {st: np.flatnonzero(perm // _L == st) for st in sts}
    gsz = len(groups[sts[0]])
    assert all(len(g) == gsz for g in groups.values()) and gsz % 16 == 0
    nvec = gsz // 16

    assert sts == list(range(sts[0], sts[0] + nst))
    ntasks = nst * D
    assert ntasks % _NW == 0 and B == _L
    tpw = ntasks // _NW

    ktab = np.zeros((nst, _L), dtype=np.int32)
    kmap = np.zeros((nst, _L), dtype=np.int32)
    for si, st in enumerate(sts):
        ks = groups[st]
        ktab[si, :gsz] = ks
        junk = gsz
        for c in range(_L):
            hits = np.flatnonzero(perm[ks] % _L == c)
            if hits.size:
                kmap[si, c] = hits[0]
            else:
                kmap[si, c] = junk
                junk += 1
    ktab = jnp.asarray(ktab)
    kmap = jnp.asarray(kmap)

    mesh = plsc.VectorSubcoreMesh(core_axis_name="c", subcore_axis_name="s")

    @functools.partial(
        pl.kernel,
        mesh=mesh,
        out_type=jax.ShapeDtypeStruct((K * D, B), jnp.float32),
        scratch_types=[
            pltpu.VMEM((nst, _L), jnp.int32),
            pltpu.VMEM((nst, _L), jnp.int32),
            pltpu.VMEM((2, _L), jnp.int32),
            pltpu.VMEM((2, _L, _L), jnp.float32),
            pltpu.VMEM((2, _L, _L), jnp.float32),
            pltpu.SemaphoreType.DMA,
            pltpu.SemaphoreType.DMA,
        ],
        compiler_params=pltpu.CompilerParams(needs_layout_passes=False),
    )
    def gather_t(x_hbm, ktab_hbm, kmap_hbm, out_hbm,
                 ktab_v, kmap_v, idx_v, stage, obuf, gsem, ssem):
        w = lax.axis_index("s") * 2 + lax.axis_index("c")
        pltpu.sync_copy(ktab_hbm, ktab_v)
        pltpu.sync_copy(kmap_hbm, kmap_v)

        iota = lax.iota(jnp.int32, 16)
        diag = [((iota + t) % 16).astype(jnp.int32) for t in range(16)]
        row_step = D * ST

        def task_params(j):
            g = w * tpw + j
            si = g // D
            d = g % D
            return si, d

        def start_gather(j, bf):
            si, d = task_params(j)
            base = d * ST + (si + sts[0])
            for jj in range(8):
                idx_v[bf, pl.ds(16 * jj, 16)] = (
                    iota * row_step + (16 * jj * row_step + base))
            return pltpu.async_copy(x_hbm.at[idx_v.at[bf]], stage.at[bf], gsem)

        def transpose(j, bf):
            si, _ = task_params(j)

            def cb_body(cb, carry):
                cols = cb * 16 + iota
                kmv = kmap_v[si, pl.ds(cb * 16, 16)]

                def rb_body(rb, carry2):
                    r0 = rb * 16
                    for t in range(16):
                        rows = r0 + diag[t]
                        v = plsc.load_gather(stage.at[bf], [rows, cols])
                        plsc.store_scatter(obuf.at[bf], [kmv, rows], v)
                    return carry2

                return lax.fori_loop(0, 8, rb_body, carry)

            lax.fori_loop(0, 8, cb_body, 0)

        def start_scatter(j, bf):
            si, d = task_params(j)
            cps = []
            for i in range(nvec):
                rows = ktab_v[si, pl.ds(16 * i, 16)] * D + d
                cps.append(pltpu.async_copy(
                    obuf.at[bf, pl.ds(16 * i, 16)], out_hbm.at[rows], ssem))
            return cps

        def wait_gather(bf):
            pltpu.make_async_copy(
                x_hbm.at[pl.ds(0, _L)], stage.at[bf], gsem).wait()

        def wait_scatters(bf):
            pltpu.make_async_copy(
                x_hbm.at[pl.ds(0, gsz)], obuf.at[bf, pl.ds(0, gsz)],
                ssem).wait()

        start_gather(0, 0)
        start_gather(1, 1)
        wait_gather(0)
        transpose(0, 0)
        start_scatter(0, 0)
        start_gather(2 % tpw, 0)
        wait_gather(1)
        transpose(1, 1)
        start_scatter(1, 1)

        def pair_body(p, carry):
            start_gather(2 * p + 1, 1)
            wait_gather(0)
            wait_scatters(0)
            transpose(2 * p, 0)
            start_scatter(2 * p, 0)
            start_gather((2 * p + 2) % tpw, 0)
            wait_gather(1)
            wait_scatters(1)
            transpose(2 * p + 1, 1)
            start_scatter(2 * p + 1, 1)
            return carry

        lax.fori_loop(1, tpw // 2, pair_body, 0)
        wait_gather(0)
        wait_scatters(0)
        wait_scatters(1)

    xv = jnp.transpose(x, (0, 2, 1)).reshape(B * D * ST, _L)
    o_view = gather_t(xv, ktab, kmap)
    return o_view.reshape(K, D, B).transpose(2, 0, 1)
